# baseline (speedup 1.0000x reference)
import functools

import jax
import jax.numpy as jnp
from jax import lax
from jax.experimental import pallas as pl
from jax.experimental.pallas import tpu as pltpu
from jax.experimental.pallas import tpu_sc as plsc

_VOCAB = 1000
_VPAD = 1024
_BLOCK = 50
_TPAD = 64
_N_EMBED = 64
_B = 1024
_T = 50

_INFO = plsc.get_sparse_core_info()
_NC = _INFO.num_cores
_NS = _INFO.num_subcores
_NW = _NC * _NS
_NV = _VPAD // _NW
_NPAIR = _T // 2


def _precompute_body(tok_emb_ref, pos_emb_ref, w_ref, b_ref,
                     fusedt_ref, poswt_ref):
    w = w_ref[...]
    dn = (((0,), (1,)), ((), ()))
    fusedt_ref[...] = (
        lax.dot_general(w, tok_emb_ref[...], dn,
                        preferred_element_type=jnp.float32)
        + b_ref[...]
    )
    poswt_ref[...] = lax.dot_general(w, pos_emb_ref[...], dn,
                                     preferred_element_type=jnp.float32)


def _precompute(tok_emb_pad, pos_emb_pad, w_pad, b_col):
    return pl.pallas_call(
        _precompute_body,
        out_shape=[
            jax.ShapeDtypeStruct((_VPAD, _VPAD), jnp.float32),
            jax.ShapeDtypeStruct((_VPAD, _TPAD), jnp.float32),
        ],
    )(tok_emb_pad, pos_emb_pad, w_pad, b_col)


@functools.partial(
    pl.kernel,
    mesh=plsc.VectorSubcoreMesh(core_axis_name="c", subcore_axis_name="s"),
    out_type=jax.ShapeDtypeStruct((_T, _VOCAB, _B), jnp.float32),
    scratch_types=[
        pltpu.VMEM((_NV, _VPAD), jnp.float32),
        pltpu.VMEM((_NV, _TPAD), jnp.float32),
        pltpu.VMEM((_B,), jnp.int32),
        pltpu.VMEM((_B,), jnp.int32),
        pltpu.VMEM((_NV, _B), jnp.float32),
        pltpu.VMEM((_NV, _B), jnp.float32),
        pltpu.VMEM((_NV, 16), jnp.float32),
        pltpu.SemaphoreType.DMA,
        pltpu.SemaphoreType.DMA,
        pltpu.SemaphoreType.DMA,
        pltpu.SemaphoreType.DMA,
    ],
    compiler_params=pltpu.CompilerParams(needs_layout_passes=False),
)
def _sc_logits(tokenst_hbm, fusedt_hbm, poswt_hbm, outt_hbm,
               fusedt_v, poswt_v, tk0, tk1, ob0, ob1, psplat_v,
               ts0, ts1, ws0, ws1):
    wid = lax.axis_index("s") * _NC + lax.axis_index("c")
    vbase = wid * _NV
    pltpu.sync_copy(fusedt_hbm.at[pl.ds(vbase, _NV)], fusedt_v)
    pltpu.sync_copy(poswt_hbm.at[pl.ds(vbase, _NV)], poswt_v)
    tks = (tk0, tk1)
    obs = (ob0, ob1)
    tss = (ts0, ts1)
    wss = (ws0, ws1)
    pltpu.async_copy(tokenst_hbm.at[0], tk0, ts0)
    pltpu.async_copy(tokenst_hbm.at[1], tk1, ts1)

    def write_out(ob, t, ws):
        @pl.when(wid < _NW - 1)
        def _():
            pltpu.async_copy(ob, outt_hbm.at[t, pl.ds(vbase, _NV)], ws)

        @pl.when(wid == _NW - 1)
        def _():
            pltpu.async_copy(ob.at[pl.ds(0, 8)],
                             outt_hbm.at[t, pl.ds(vbase, 8)], ws)

    def drain_write(ob, t, ws):
        @pl.when(wid < _NW - 1)
        def _():
            pltpu.make_async_copy(
                ob, outt_hbm.at[t, pl.ds(vbase, _NV)], ws).wait()

        @pl.when(wid == _NW - 1)
        def _():
            pltpu.make_async_copy(
                ob.at[pl.ds(0, 8)], outt_hbm.at[t, pl.ds(vbase, 8)], ws).wait()

    def pair_body(p, carry):
        for par in (0, 1):
            t = 2 * p + par
            tk, ob, ts, ws = tks[par], obs[par], tss[par], wss[par]
            pltpu.make_async_copy(tokenst_hbm.at[t], tk, ts).wait()

            @pl.when(p > 0)
            def _():
                drain_write(ob, t, ws)

            tvec = jnp.zeros((16,), jnp.int32) + t
            for vl in range(_NV):
                psplat_v[vl, :] = plsc.load_gather(
                    poswt_v, [jnp.full((16,), vl, jnp.int32), tvec])

            def bc_body(bc, bcarry):
                tok = tk[pl.ds(16 * bc, 16)]
                for vl in range(_NV):
                    vals = plsc.load_gather(
                        fusedt_v, [jnp.full((16,), vl, jnp.int32), tok])
                    ob[vl, pl.ds(16 * bc, 16)] = vals + psplat_v[vl, :]
                return bcarry

            lax.fori_loop(0, _B // 16, bc_body, 0)
            write_out(ob, t, ws)

            @pl.when(p < _NPAIR - 1)
            def _():
                pltpu.async_copy(tokenst_hbm.at[t + 2], tk, ts)
        return carry

    lax.fori_loop(0, _NPAIR, pair_body, 0)
    for par in (0, 1):
        drain_write(obs[par], _T - 2 + par, wss[par])


def kernel(tokens, tok_emb, pos_emb, W, b):
    w_pad = jnp.pad(W, ((0, 0), (0, _VPAD - _VOCAB)))
    b_col = jnp.pad(b, (0, _VPAD - _VOCAB)).reshape(_VPAD, 1)
    tok_emb_pad = jnp.pad(tok_emb, ((0, _VPAD - _VOCAB), (0, 0)))
    pos_emb_pad = jnp.pad(pos_emb, ((0, _TPAD - _BLOCK), (0, 0)))
    tokenst = tokens.astype(jnp.int32).T
    fusedt, poswt = _precompute(tok_emb_pad, pos_emb_pad, w_pad, b_col)
    outt = _sc_logits(tokenst, fusedt, poswt)
    return jnp.transpose(outt, (2, 0, 1))


# speedup vs baseline: 3.2155x; 3.2155x over previous
import functools

import jax
import jax.numpy as jnp
from jax import lax
from jax.experimental import pallas as pl
from jax.experimental.pallas import tpu as pltpu
from jax.experimental.pallas import tpu_sc as plsc

_VOCAB = 1000
_VPAD = 1024
_BLOCK = 50
_TPAD = 64
_N_EMBED = 64
_B = 1024
_T = 50

_INFO = plsc.get_sparse_core_info()
_NC = _INFO.num_cores
_NS = _INFO.num_subcores
_NW = _NC * _NS
_NV = _VPAD // _NW
_NPAIR = _T // 2


def _precompute_body(tok_emb_ref, pos_emb_ref, w_ref, b_ref,
                     fusedt_ref, poswt_ref):
    w = w_ref[...]
    dn = (((0,), (1,)), ((), ()))
    fusedt_ref[...] = (
        lax.dot_general(w, tok_emb_ref[...], dn,
                        preferred_element_type=jnp.float32)
        + b_ref[...]
    )
    poswt_ref[...] = lax.dot_general(w, pos_emb_ref[...], dn,
                                     preferred_element_type=jnp.float32)


def _precompute(tok_emb_pad, pos_emb_pad, w_pad, b_col):
    return pl.pallas_call(
        _precompute_body,
        out_shape=[
            jax.ShapeDtypeStruct((_VPAD, _VPAD), jnp.float32),
            jax.ShapeDtypeStruct((_VPAD, _TPAD), jnp.float32),
        ],
    )(tok_emb_pad, pos_emb_pad, w_pad, b_col)


@functools.partial(
    pl.kernel,
    mesh=plsc.VectorSubcoreMesh(core_axis_name="c", subcore_axis_name="s"),
    out_type=jax.ShapeDtypeStruct((_T, _VOCAB, _B), jnp.float32),
    scratch_types=[
        pltpu.VMEM((_NV, _VPAD), jnp.float32),
        pltpu.VMEM((_NV, _TPAD), jnp.float32),
        pltpu.VMEM((_B,), jnp.int32),
        pltpu.VMEM((_B,), jnp.int32),
        pltpu.VMEM((_NV, _B), jnp.float32),
        pltpu.VMEM((_NV, _B), jnp.float32),
        pltpu.VMEM((_NV, 16), jnp.float32),
        pltpu.SemaphoreType.DMA,
        pltpu.SemaphoreType.DMA,
        pltpu.SemaphoreType.DMA,
        pltpu.SemaphoreType.DMA,
    ],
    compiler_params=pltpu.CompilerParams(needs_layout_passes=False),
)
def _sc_logits(tokenst_hbm, fusedt_hbm, poswt_hbm, outt_hbm,
               fusedt_v, poswt_v, tk0, tk1, ob0, ob1, psplat_v,
               ts0, ts1, ws0, ws1):
    wid = lax.axis_index("s") * _NC + lax.axis_index("c")
    vbase = wid * _NV
    pltpu.sync_copy(fusedt_hbm.at[pl.ds(vbase, _NV)], fusedt_v)
    pltpu.sync_copy(poswt_hbm.at[pl.ds(vbase, _NV)], poswt_v)
    tks = (tk0, tk1)
    obs = (ob0, ob1)
    tss = (ts0, ts1)
    wss = (ws0, ws1)
    pltpu.async_copy(tokenst_hbm.at[0], tk0, ts0)
    pltpu.async_copy(tokenst_hbm.at[1], tk1, ts1)

    def write_out(ob, t, ws):
        @pl.when(wid < _NW - 1)
        def _():
            pltpu.async_copy(ob, outt_hbm.at[t, pl.ds(vbase, _NV)], ws)

        @pl.when(wid == _NW - 1)
        def _():
            pltpu.async_copy(ob.at[pl.ds(0, 8)],
                             outt_hbm.at[t, pl.ds(vbase, 8)], ws)

    def drain_write(ob, t, ws):
        @pl.when(wid < _NW - 1)
        def _():
            pltpu.make_async_copy(
                ob, outt_hbm.at[t, pl.ds(vbase, _NV)], ws).wait()

        @pl.when(wid == _NW - 1)
        def _():
            pltpu.make_async_copy(
                ob.at[pl.ds(0, 8)], outt_hbm.at[t, pl.ds(vbase, 8)], ws).wait()

    def pair_body(p, carry):
        for par in (0, 1):
            t = 2 * p + par
            tk, ob, ts, ws = tks[par], obs[par], tss[par], wss[par]
            pltpu.make_async_copy(tokenst_hbm.at[t], tk, ts).wait()

            @pl.when(p > 0)
            def _():
                drain_write(ob, t, ws)

            tvec = jnp.zeros((16,), jnp.int32) + t
            for vl in range(_NV):
                psplat_v[vl, :] = plsc.load_gather(
                    poswt_v, [jnp.full((16,), vl, jnp.int32), tvec])

            @plsc.parallel_loop(0, _B // 16)
            def bc_body(bc):
                tok = tk[pl.ds(16 * bc, 16)]
                for vl in range(_NV):
                    vals = plsc.load_gather(
                        fusedt_v, [jnp.full((16,), vl, jnp.int32), tok])
                    ob[vl, pl.ds(16 * bc, 16)] = vals + psplat_v[vl, :]
            write_out(ob, t, ws)

            @pl.when(p < _NPAIR - 1)
            def _():
                pltpu.async_copy(tokenst_hbm.at[t + 2], tk, ts)
        return carry

    lax.fori_loop(0, _NPAIR, pair_body, 0)
    for par in (0, 1):
        drain_write(obs[par], _T - 2 + par, wss[par])


def kernel(tokens, tok_emb, pos_emb, W, b):
    w_pad = jnp.pad(W, ((0, 0), (0, _VPAD - _VOCAB)))
    b_col = jnp.pad(b, (0, _VPAD - _VOCAB)).reshape(_VPAD, 1)
    tok_emb_pad = jnp.pad(tok_emb, ((0, _VPAD - _VOCAB), (0, 0)))
    pos_emb_pad = jnp.pad(pos_emb, ((0, _TPAD - _BLOCK), (0, 0)))
    tokenst = tokens.astype(jnp.int32).T
    fusedt, poswt = _precompute(tok_emb_pad, pos_emb_pad, w_pad, b_col)
    outt = _sc_logits(tokenst, fusedt, poswt)
    return jnp.transpose(outt, (2, 0, 1))


# speedup vs baseline: 4.4221x; 1.3753x over previous
import functools

import jax
import jax.numpy as jnp
from jax import lax
from jax.experimental import pallas as pl
from jax.experimental.pallas import tpu as pltpu
from jax.experimental.pallas import tpu_sc as plsc

_VOCAB = 1000
_VPAD = 1024
_BLOCK = 50
_TPAD = 64
_N_EMBED = 64
_B = 1024
_T = 50

_INFO = plsc.get_sparse_core_info()
_NC = _INFO.num_cores
_NS = _INFO.num_subcores
_NW = _NC * _NS
_NV = _VPAD // _NW
_NPAIR = _T // 2


def _precompute_body(tok_emb_ref, pos_emb_ref, w_ref, b_ref,
                     fusedt_ref, poswt_ref):
    w = w_ref[...]
    dn = (((0,), (1,)), ((), ()))
    fusedt_ref[...] = (
        lax.dot_general(w, tok_emb_ref[...], dn,
                        preferred_element_type=jnp.float32)
        + b_ref[...]
    )
    poswt_ref[...] = lax.dot_general(w, pos_emb_ref[...], dn,
                                     preferred_element_type=jnp.float32)


def _precompute(tok_emb_pad, pos_emb_pad, w_pad, b_col):
    return pl.pallas_call(
        _precompute_body,
        out_shape=[
            jax.ShapeDtypeStruct((_VPAD, _VPAD), jnp.float32),
            jax.ShapeDtypeStruct((_VPAD, _TPAD), jnp.float32),
        ],
    )(tok_emb_pad, pos_emb_pad, w_pad, b_col)


@functools.partial(
    pl.kernel,
    mesh=plsc.VectorSubcoreMesh(core_axis_name="c", subcore_axis_name="s"),
    out_type=jax.ShapeDtypeStruct((_T, _VOCAB, _B), jnp.float32),
    scratch_types=[
        pltpu.VMEM((_NV, _VPAD), jnp.float32),
        pltpu.VMEM((_NV, _TPAD), jnp.float32),
        pltpu.VMEM((_B,), jnp.int32),
        pltpu.VMEM((_B,), jnp.int32),
        pltpu.VMEM((_NV, _B), jnp.float32),
        pltpu.VMEM((_NV, _B), jnp.float32),
        pltpu.VMEM((_NV, 16), jnp.float32),
        pltpu.SemaphoreType.DMA,
        pltpu.SemaphoreType.DMA,
        pltpu.SemaphoreType.DMA,
        pltpu.SemaphoreType.DMA,
    ],
    compiler_params=pltpu.CompilerParams(needs_layout_passes=False),
)
def _sc_logits(tokenst_hbm, fusedt_hbm, poswt_hbm, outt_hbm,
               fusedt_v, poswt_v, tk0, tk1, ob0, ob1, psplat_v,
               ts0, ts1, ws0, ws1):
    wid = lax.axis_index("s") * _NC + lax.axis_index("c")
    vbase = wid * _NV
    pltpu.sync_copy(fusedt_hbm.at[pl.ds(vbase, _NV)], fusedt_v)
    pltpu.sync_copy(poswt_hbm.at[pl.ds(vbase, _NV)], poswt_v)
    tks = (tk0, tk1)
    obs = (ob0, ob1)
    tss = (ts0, ts1)
    wss = (ws0, ws1)
    pltpu.async_copy(tokenst_hbm.at[0], tk0, ts0)
    pltpu.async_copy(tokenst_hbm.at[1], tk1, ts1)

    def write_out(ob, t, ws):
        @pl.when(wid < _NW - 1)
        def _():
            pltpu.async_copy(ob, outt_hbm.at[t, pl.ds(vbase, _NV)], ws)

        @pl.when(wid == _NW - 1)
        def _():
            pltpu.async_copy(ob.at[pl.ds(0, 8)],
                             outt_hbm.at[t, pl.ds(vbase, 8)], ws)

    def drain_write(ob, t, ws):
        @pl.when(wid < _NW - 1)
        def _():
            pltpu.make_async_copy(
                ob, outt_hbm.at[t, pl.ds(vbase, _NV)], ws).wait()

        @pl.when(wid == _NW - 1)
        def _():
            pltpu.make_async_copy(
                ob.at[pl.ds(0, 8)], outt_hbm.at[t, pl.ds(vbase, 8)], ws).wait()

    def pair_body(p, carry):
        for par in (0, 1):
            t = 2 * p + par
            tk, ob, ts, ws = tks[par], obs[par], tss[par], wss[par]
            pltpu.make_async_copy(tokenst_hbm.at[t], tk, ts).wait()

            @pl.when(p > 0)
            def _():
                drain_write(ob, t, ws)

            tvec = jnp.zeros((16,), jnp.int32) + t
            for vg in range(_NV // 8):
                psplat = [
                    plsc.load_gather(
                        poswt_v,
                        [jnp.full((16,), 8 * vg + k, jnp.int32), tvec])
                    for k in range(8)
                ]

                @plsc.parallel_loop(0, _B // 16)
                def bc_body(bc):
                    tok = tk[pl.ds(16 * bc, 16)]
                    for k in range(8):
                        vl = 8 * vg + k
                        vals = plsc.load_gather(
                            fusedt_v, [jnp.full((16,), vl, jnp.int32), tok])
                        ob[vl, pl.ds(16 * bc, 16)] = vals + psplat[k]
            write_out(ob, t, ws)

            @pl.when(p < _NPAIR - 1)
            def _():
                pltpu.async_copy(tokenst_hbm.at[t + 2], tk, ts)
        return carry

    lax.fori_loop(0, _NPAIR, pair_body, 0)
    for par in (0, 1):
        drain_write(obs[par], _T - 2 + par, wss[par])


def kernel(tokens, tok_emb, pos_emb, W, b):
    w_pad = jnp.pad(W, ((0, 0), (0, _VPAD - _VOCAB)))
    b_col = jnp.pad(b, (0, _VPAD - _VOCAB)).reshape(_VPAD, 1)
    tok_emb_pad = jnp.pad(tok_emb, ((0, _VPAD - _VOCAB), (0, 0)))
    pos_emb_pad = jnp.pad(pos_emb, ((0, _TPAD - _BLOCK), (0, 0)))
    tokenst = tokens.astype(jnp.int32).T
    fusedt, poswt = _precompute(tok_emb_pad, pos_emb_pad, w_pad, b_col)
    outt = _sc_logits(tokenst, fusedt, poswt)
    return jnp.transpose(outt, (2, 0, 1))
